# baseline (speedup 1.0000x reference)
import functools

import jax
import jax.numpy as jnp
from jax.experimental import pallas as pl
from jax.experimental.pallas import tpu as pltpu

_PASSES = 3


def _ggnn_body(h0_ref, adj_ref, wmsg_ref, bmsg_ref, wih_ref, whh_ref,
               bih_ref, bhh_ref, out_ref, msgb_ref, h_ref, packed_ref,
               abf_ref, *, T, BR, R, D, M, G):
    p = pl.program_id(0)
    r = pl.program_id(1)

    @pl.when(jnp.logical_and(p == 0, r == 0))
    def _init():
        h_ref[...] = h0_ref[...]

    @pl.when(r == 0)
    def _messages():
        h_cur = h_ref[...]
        for i in range(T):
            m = (jnp.dot(h_cur, wmsg_ref[i], preferred_element_type=jnp.float32)
                 + bmsg_ref[i])
            msgb_ref[i] = m.astype(jnp.bfloat16)

    rows = pl.ds(r * BR, BR)
    acc = jnp.zeros((BR, M), dtype=jnp.float32)
    for i in range(T):
        @pl.when(p == 0)
        def _load_and_pack(i=i):
            a = adj_ref[i]
            word = jnp.zeros((BR, 128), dtype=jnp.int32)
            for g in range(G):
                bit = (a[:, g * 128:(g + 1) * 128] != 0.0).astype(jnp.int32)
                word = word | (bit << g)
            packed_ref[i, rows, :] = word
            abf_ref[...] = a.astype(jnp.bfloat16)

        @pl.when(p != 0)
        def _expand(i=i):
            word = packed_ref[i, rows, :]
            parts = [((word >> g) & 1).astype(jnp.bfloat16) for g in range(G)]
            abf_ref[...] = jnp.concatenate(parts, axis=1)

        acc = acc + jnp.dot(abf_ref[...], msgb_ref[i],
                            preferred_element_type=jnp.float32)

    x = jnp.maximum(acc, 0.0)
    h = h_ref[rows, :]
    gi = jnp.dot(x, wih_ref[...], preferred_element_type=jnp.float32) + bih_ref[...]
    gh = jnp.dot(h, whh_ref[...], preferred_element_type=jnp.float32) + bhh_ref[...]
    i_r, i_z, i_n = gi[:, :D], gi[:, D:2 * D], gi[:, 2 * D:]
    h_r, h_z, h_n = gh[:, :D], gh[:, D:2 * D], gh[:, 2 * D:]
    rg = jax.nn.sigmoid(i_r + h_r)
    zg = jax.nn.sigmoid(i_z + h_z)
    ng = jnp.tanh(i_n + rg * h_n)
    h_new = (1.0 - zg) * ng + zg * h
    h_ref[rows, :] = h_new
    out_ref[rows, :] = h_new


def kernel(h_node, adjacency, W_msg, b_msg, W_ih, W_hh, b_ih, b_hh):
    N, D = h_node.shape
    T = adjacency.shape[0]
    M = W_msg.shape[1]
    Gate = 3 * D
    BR = 128 if N % 128 == 0 else N
    R = N // BR
    G = N // 128

    grid = (_PASSES, R)
    in_specs = [
        pl.BlockSpec((N, D), lambda p, r: (0, 0)),
        pl.BlockSpec((T, BR, N), lambda p, r: (0, jnp.where(p == 0, r, R - 1), 0)),
        pl.BlockSpec((T, D, M), lambda p, r: (0, 0, 0)),
        pl.BlockSpec((T, 1, M), lambda p, r: (0, 0, 0)),
        pl.BlockSpec((M, Gate), lambda p, r: (0, 0)),
        pl.BlockSpec((D, Gate), lambda p, r: (0, 0)),
        pl.BlockSpec((1, Gate), lambda p, r: (0, 0)),
        pl.BlockSpec((1, Gate), lambda p, r: (0, 0)),
    ]
    out_specs = pl.BlockSpec((N, D), lambda p, r: (0, 0))
    scratch_shapes = [
        pltpu.VMEM((T, N, M), jnp.bfloat16),
        pltpu.VMEM((N, D), jnp.float32),
        pltpu.VMEM((T, N, 128), jnp.int32),
        pltpu.VMEM((BR, N), jnp.bfloat16),
    ]

    f = pl.pallas_call(
        functools.partial(_ggnn_body, T=T, BR=BR, R=R, D=D, M=M, G=G),
        grid=grid,
        in_specs=in_specs,
        out_specs=out_specs,
        out_shape=jax.ShapeDtypeStruct((N, D), jnp.float32),
        scratch_shapes=scratch_shapes,
        compiler_params=pltpu.CompilerParams(
            dimension_semantics=("arbitrary", "arbitrary")),
    )
    return f(h_node, adjacency, jnp.transpose(W_msg, (0, 2, 1)),
             b_msg.reshape(T, 1, M), W_ih.T, W_hh.T,
             b_ih.reshape(1, Gate), b_hh.reshape(1, Gate))


# speedup vs baseline: 2.0021x; 2.0021x over previous
import functools

import jax
import jax.numpy as jnp
from jax.experimental import pallas as pl
from jax.experimental.pallas import tpu as pltpu

_PASSES = 3


def _bit(g):
    return jnp.int32(-2147483648) if g == 31 else jnp.int32(1 << g)


def _ggnn_body(h0_ref, adj_ref, wmsg_ref, bmsg_ref, wih_ref, whh_ref,
               bih_ref, bhh_ref, out_ref, msgb_ref, h_ref, packed_ref,
               acc_ref, *, T, BR, R, D, M, G):
    p = pl.program_id(0)
    r = pl.program_id(1)

    @pl.when(jnp.logical_and(p == 0, r == 0))
    def _init():
        h_ref[...] = h0_ref[...]

    @pl.when(r == 0)
    def _messages():
        h_cur = h_ref[...]
        for i in range(T):
            m = (jnp.dot(h_cur, wmsg_ref[i], preferred_element_type=jnp.float32)
                 + bmsg_ref[i])
            msgb_ref[i] = m.astype(jnp.bfloat16)

    rows = pl.ds(r * BR, BR)

    @pl.when(p == 0)
    def _agg_pass0():
        acc = jnp.zeros((BR, M), dtype=jnp.float32)
        for i in range(T):
            a = adj_ref[i]
            word = jnp.zeros((BR, 128), dtype=jnp.int32)
            for g in range(G):
                nz = a[:, g * 128:(g + 1) * 128] != 0.0
                word = word | jnp.where(nz, _bit(g), jnp.int32(0))
            packed_ref[i, rows, :] = word
            acc = acc + jnp.dot(a.astype(jnp.bfloat16), msgb_ref[i],
                                preferred_element_type=jnp.float32)
        acc_ref[...] = acc

    @pl.when(p != 0)
    def _agg_expand():
        acc = jnp.zeros((BR, M), dtype=jnp.float32)
        for i in range(T):
            word = packed_ref[i, rows, :]
            parts = []
            for g in range(G):
                bit = (word & _bit(g)) != 0
                parts.append(jnp.where(bit, jnp.float32(1.0), jnp.float32(0.0)))
            a_bf = jnp.concatenate(parts, axis=1).astype(jnp.bfloat16)
            acc = acc + jnp.dot(a_bf, msgb_ref[i],
                                preferred_element_type=jnp.float32)
        acc_ref[...] = acc

    x = jnp.maximum(acc_ref[...], 0.0)
    h = h_ref[rows, :]
    gi = jnp.dot(x, wih_ref[...], preferred_element_type=jnp.float32) + bih_ref[...]
    gh = jnp.dot(h, whh_ref[...], preferred_element_type=jnp.float32) + bhh_ref[...]
    i_r, i_z, i_n = gi[:, :D], gi[:, D:2 * D], gi[:, 2 * D:]
    h_r, h_z, h_n = gh[:, :D], gh[:, D:2 * D], gh[:, 2 * D:]
    rg = jax.nn.sigmoid(i_r + h_r)
    zg = jax.nn.sigmoid(i_z + h_z)
    ng = jnp.tanh(i_n + rg * h_n)
    h_new = (1.0 - zg) * ng + zg * h
    h_ref[rows, :] = h_new
    out_ref[rows, :] = h_new


def kernel(h_node, adjacency, W_msg, b_msg, W_ih, W_hh, b_ih, b_hh):
    N, D = h_node.shape
    T = adjacency.shape[0]
    M = W_msg.shape[1]
    Gate = 3 * D
    BR = 128 if N % 128 == 0 else N
    R = N // BR
    G = N // 128

    grid = (_PASSES, R)
    in_specs = [
        pl.BlockSpec((N, D), lambda p, r: (0, 0)),
        pl.BlockSpec((T, BR, N), lambda p, r: (0, jnp.where(p == 0, r, R - 1), 0)),
        pl.BlockSpec((T, D, M), lambda p, r: (0, 0, 0)),
        pl.BlockSpec((T, 1, M), lambda p, r: (0, 0, 0)),
        pl.BlockSpec((M, Gate), lambda p, r: (0, 0)),
        pl.BlockSpec((D, Gate), lambda p, r: (0, 0)),
        pl.BlockSpec((1, Gate), lambda p, r: (0, 0)),
        pl.BlockSpec((1, Gate), lambda p, r: (0, 0)),
    ]
    out_specs = pl.BlockSpec((N, D), lambda p, r: (0, 0))
    scratch_shapes = [
        pltpu.VMEM((T, N, M), jnp.bfloat16),
        pltpu.VMEM((N, D), jnp.float32),
        pltpu.VMEM((T, N, 128), jnp.int32),
        pltpu.VMEM((BR, M), jnp.float32),
    ]

    f = pl.pallas_call(
        functools.partial(_ggnn_body, T=T, BR=BR, R=R, D=D, M=M, G=G),
        grid=grid,
        in_specs=in_specs,
        out_specs=out_specs,
        out_shape=jax.ShapeDtypeStruct((N, D), jnp.float32),
        scratch_shapes=scratch_shapes,
        compiler_params=pltpu.CompilerParams(
            dimension_semantics=("arbitrary", "arbitrary")),
    )
    return f(h_node, adjacency, jnp.transpose(W_msg, (0, 2, 1)),
             b_msg.reshape(T, 1, M), W_ih.T, W_hh.T,
             b_ih.reshape(1, Gate), b_hh.reshape(1, Gate))


# speedup vs baseline: 2.1675x; 1.0826x over previous
import functools

import jax
import jax.numpy as jnp
from jax.experimental import pallas as pl
from jax.experimental.pallas import tpu as pltpu

_PASSES = 3


def _bit(g):
    return jnp.int32(-2147483648) if g == 31 else jnp.int32(1 << g)


def _ggnn_body(h0_ref, adj_ref, wmsg_ref, bmsg_ref, wih_ref, whh_ref,
               bih_ref, bhh_ref, out_ref, msgb_ref, h_ref, packed_ref,
               acc_ref, *, T, BR, R, D, M, G):
    p = pl.program_id(0)
    r = pl.program_id(1)

    @pl.when(jnp.logical_and(p == 0, r == 0))
    def _init():
        h_ref[...] = h0_ref[...]

    @pl.when(r == 0)
    def _messages():
        h_cur = h_ref[...]
        for i in range(T):
            m = (jnp.dot(h_cur, wmsg_ref[i], preferred_element_type=jnp.float32)
                 + bmsg_ref[i])
            msgb_ref[i] = m.astype(jnp.bfloat16)

    rows = pl.ds(r * BR, BR)

    @pl.when(p == 0)
    def _agg_pass0():
        acc = jnp.zeros((BR, M), dtype=jnp.float32)
        for i in range(T):
            a = adj_ref[i]
            word = jnp.zeros((BR, 128), dtype=jnp.int32)
            for g in range(G):
                nz = a[:, g * 128:(g + 1) * 128] != 0.0
                word = word | jnp.where(nz, _bit(g), jnp.int32(0))
            packed_ref[i, rows, :] = word
            acc = acc + jnp.dot(a.astype(jnp.bfloat16), msgb_ref[i],
                                preferred_element_type=jnp.float32)
        acc_ref[...] = acc

    @pl.when(p != 0)
    def _agg_expand():
        acc = jnp.zeros((BR, M), dtype=jnp.float32)
        for i in range(T):
            word = packed_ref[i, rows, :]
            parts = []
            for g in range(G):
                bit = (word & _bit(g)) != 0
                parts.append(jnp.where(bit, jnp.float32(1.0), jnp.float32(0.0)))
            a_bf = jnp.concatenate(parts, axis=1).astype(jnp.bfloat16)
            acc = acc + jnp.dot(a_bf, msgb_ref[i],
                                preferred_element_type=jnp.float32)
        acc_ref[...] = acc

    x = jnp.maximum(acc_ref[...], 0.0)
    h = h_ref[rows, :]
    gi = jnp.dot(x, wih_ref[...], preferred_element_type=jnp.float32) + bih_ref[...]
    gh = jnp.dot(h, whh_ref[...], preferred_element_type=jnp.float32) + bhh_ref[...]
    i_r, i_z, i_n = gi[:, :D], gi[:, D:2 * D], gi[:, 2 * D:]
    h_r, h_z, h_n = gh[:, :D], gh[:, D:2 * D], gh[:, 2 * D:]
    rg = jax.nn.sigmoid(i_r + h_r)
    zg = jax.nn.sigmoid(i_z + h_z)
    ng = jnp.tanh(i_n + rg * h_n)
    h_new = (1.0 - zg) * ng + zg * h
    h_ref[rows, :] = h_new
    out_ref[rows, :] = h_new


def kernel(h_node, adjacency, W_msg, b_msg, W_ih, W_hh, b_ih, b_hh):
    N, D = h_node.shape
    T = adjacency.shape[0]
    M = W_msg.shape[1]
    Gate = 3 * D
    BR = 256 if N % 256 == 0 else (128 if N % 128 == 0 else N)
    R = N // BR
    G = N // 128

    grid = (_PASSES, R)
    in_specs = [
        pl.BlockSpec((N, D), lambda p, r: (0, 0)),
        pl.BlockSpec((T, BR, N), lambda p, r: (0, jnp.where(p == 0, r, R - 1), 0)),
        pl.BlockSpec((T, D, M), lambda p, r: (0, 0, 0)),
        pl.BlockSpec((T, 1, M), lambda p, r: (0, 0, 0)),
        pl.BlockSpec((M, Gate), lambda p, r: (0, 0)),
        pl.BlockSpec((D, Gate), lambda p, r: (0, 0)),
        pl.BlockSpec((1, Gate), lambda p, r: (0, 0)),
        pl.BlockSpec((1, Gate), lambda p, r: (0, 0)),
    ]
    out_specs = pl.BlockSpec((N, D), lambda p, r: (0, 0))
    scratch_shapes = [
        pltpu.VMEM((T, N, M), jnp.bfloat16),
        pltpu.VMEM((N, D), jnp.float32),
        pltpu.VMEM((T, N, 128), jnp.int32),
        pltpu.VMEM((BR, M), jnp.float32),
    ]

    f = pl.pallas_call(
        functools.partial(_ggnn_body, T=T, BR=BR, R=R, D=D, M=M, G=G),
        grid=grid,
        in_specs=in_specs,
        out_specs=out_specs,
        out_shape=jax.ShapeDtypeStruct((N, D), jnp.float32),
        scratch_shapes=scratch_shapes,
        compiler_params=pltpu.CompilerParams(
            dimension_semantics=("arbitrary", "arbitrary")),
    )
    return f(h_node, adjacency, jnp.transpose(W_msg, (0, 2, 1)),
             b_msg.reshape(T, 1, M), W_ih.T, W_hh.T,
             b_ih.reshape(1, Gate), b_hh.reshape(1, Gate))


# speedup vs baseline: 2.1826x; 1.0070x over previous
import functools

import jax
import jax.numpy as jnp
from jax.experimental import pallas as pl
from jax.experimental.pallas import tpu as pltpu

_PASSES = 3


def _bit(g):
    return jnp.int32(-2147483648) if g == 31 else jnp.int32(1 << g)


def _ggnn_body(h0_ref, adj_ref, wstack_ref, bmsg_ref, wih_ref, whh_ref,
               bih_ref, bhh_ref, out_ref, h_ref, hx_ref, packed_ref,
               bias_ref, acc_ref, *, T, BR, R, D, M, G, E):
    p = pl.program_id(0)
    r = pl.program_id(1)

    @pl.when(jnp.logical_and(p == 0, r == 0))
    def _init():
        h_ref[...] = h0_ref[...]
        hx_ref[:, :D] = h0_ref[...].astype(jnp.bfloat16)
        lane = jax.lax.broadcasted_iota(jnp.int32, (hx_ref.shape[0], E - D), 1)
        hx_ref[:, D:] = jnp.where(lane == 0, jnp.float32(1.0),
                                  jnp.float32(0.0)).astype(jnp.bfloat16)

    @pl.when(jnp.logical_and(p != 0, r == 0))
    def _refresh_h():
        hx_ref[:, :D] = h_ref[...].astype(jnp.bfloat16)

    rows = pl.ds(r * BR, BR)

    @pl.when(p == 0)
    def _agg_pass0():
        acc = jnp.zeros((BR, M), dtype=jnp.float32)
        bias = jnp.zeros((BR, M), dtype=jnp.float32)
        s_parts = []
        for i in range(T):
            a = adj_ref[i]
            word = jnp.zeros((BR, 128), dtype=jnp.int32)
            for g in range(G):
                nz = a[:, g * 128:(g + 1) * 128] != 0.0
                word = word | jnp.where(nz, _bit(g), jnp.int32(0))
            packed_ref[i, rows, :] = word
            ri = jnp.dot(a.astype(jnp.bfloat16), hx_ref[...],
                         preferred_element_type=jnp.float32)
            s_parts.append(ri[:, :D])
            bias = bias + ri[:, D:D + 1] * bmsg_ref[i]
        s_cat = jnp.concatenate(s_parts, axis=1)
        acc = jnp.dot(s_cat, wstack_ref[...],
                      preferred_element_type=jnp.float32) + bias
        bias_ref[rows, :] = bias
        acc_ref[...] = acc

    @pl.when(p != 0)
    def _agg_expand():
        s_parts = []
        for i in range(T):
            word = packed_ref[i, rows, :]
            parts = []
            for g in range(G):
                bitv = (word & _bit(g)) != 0
                parts.append(jnp.where(bitv, jnp.float32(1.0), jnp.float32(0.0)))
            a_bf = jnp.concatenate(parts, axis=1).astype(jnp.bfloat16)
            s_parts.append(jnp.dot(a_bf, hx_ref[:, :D],
                                   preferred_element_type=jnp.float32))
        s_cat = jnp.concatenate(s_parts, axis=1)
        acc_ref[...] = (jnp.dot(s_cat, wstack_ref[...],
                                preferred_element_type=jnp.float32)
                        + bias_ref[rows, :])

    x = jnp.maximum(acc_ref[...], 0.0)
    h = h_ref[rows, :]
    gi = jnp.dot(x, wih_ref[...], preferred_element_type=jnp.float32) + bih_ref[...]
    gh = jnp.dot(h, whh_ref[...], preferred_element_type=jnp.float32) + bhh_ref[...]
    i_r, i_z, i_n = gi[:, :D], gi[:, D:2 * D], gi[:, 2 * D:]
    h_r, h_z, h_n = gh[:, :D], gh[:, D:2 * D], gh[:, 2 * D:]
    rg = jax.nn.sigmoid(i_r + h_r)
    zg = jax.nn.sigmoid(i_z + h_z)
    ng = jnp.tanh(i_n + rg * h_n)
    h_new = (1.0 - zg) * ng + zg * h
    h_ref[rows, :] = h_new
    out_ref[rows, :] = h_new


def kernel(h_node, adjacency, W_msg, b_msg, W_ih, W_hh, b_ih, b_hh):
    N, D = h_node.shape
    T = adjacency.shape[0]
    M = W_msg.shape[1]
    Gate = 3 * D
    E = 2 * D
    BR = 256 if N % 256 == 0 else (128 if N % 128 == 0 else N)
    R = N // BR
    G = N // 128

    grid = (_PASSES, R)
    in_specs = [
        pl.BlockSpec((N, D), lambda p, r: (0, 0)),
        pl.BlockSpec((T, BR, N), lambda p, r: (0, jnp.where(p == 0, r, R - 1), 0)),
        pl.BlockSpec((T * D, M), lambda p, r: (0, 0)),
        pl.BlockSpec((T, 1, M), lambda p, r: (0, 0, 0)),
        pl.BlockSpec((M, Gate), lambda p, r: (0, 0)),
        pl.BlockSpec((D, Gate), lambda p, r: (0, 0)),
        pl.BlockSpec((1, Gate), lambda p, r: (0, 0)),
        pl.BlockSpec((1, Gate), lambda p, r: (0, 0)),
    ]
    out_specs = pl.BlockSpec((N, D), lambda p, r: (0, 0))
    scratch_shapes = [
        pltpu.VMEM((N, D), jnp.float32),
        pltpu.VMEM((N, E), jnp.bfloat16),
        pltpu.VMEM((T, N, 128), jnp.int32),
        pltpu.VMEM((N, M), jnp.float32),
        pltpu.VMEM((BR, M), jnp.float32),
    ]

    f = pl.pallas_call(
        functools.partial(_ggnn_body, T=T, BR=BR, R=R, D=D, M=M, G=G, E=E),
        grid=grid,
        in_specs=in_specs,
        out_specs=out_specs,
        out_shape=jax.ShapeDtypeStruct((N, D), jnp.float32),
        scratch_shapes=scratch_shapes,
        compiler_params=pltpu.CompilerParams(
            dimension_semantics=("arbitrary", "arbitrary")),
    )
    wstack = jnp.transpose(W_msg, (0, 2, 1)).reshape(T * D, M)
    return f(h_node, adjacency, wstack,
             b_msg.reshape(T, 1, M), W_ih.T, W_hh.T,
             b_ih.reshape(1, Gate), b_hh.reshape(1, Gate))
